# baseline (speedup 1.0000x reference)
import functools

import jax
import jax.numpy as jnp
from jax import lax
from jax.experimental import pallas as pl
from jax.experimental.pallas import tpu as pltpu
from jax.experimental.pallas import tpu_sc as plsc

N = 10000
E = 320000
LANE = 128
CT = 160
PE = 16 * CT * LANE
ROWS2D = PE // LANE
DT = ROWS2D // 32
ACC_R = 10240
DEG_R = 10112
F = 128
H = 256
RB = 1000

_mesh = plsc.VectorSubcoreMesh(core_axis_name="c", subcore_axis_name="s")


def _sc_deg(dst2d):
    @functools.partial(
        pl.kernel,
        mesh=_mesh,
        out_type=jax.ShapeDtypeStruct((2 * DEG_R,), jnp.float32),
        scratch_types=[
            pltpu.VMEM((DT, LANE), jnp.int32),
            pltpu.VMEM((LANE,), jnp.float32),
            pltpu.VMEM((LANE,), jnp.float32),
            pltpu.VMEM_SHARED((DEG_R,), jnp.float32),
            pltpu.SemaphoreType.DMA,
        ],
    )
    def k(dst_hbm, out, dst_v, ones_v, zv, acc, t0):
        c = lax.axis_index("c")
        s = lax.axis_index("s")
        one = jnp.full((16,), 1.0, dtype=jnp.float32)
        zer = jnp.zeros((16,), dtype=jnp.float32)
        for j in range(LANE // 16):
            ones_v[pl.ds(16 * j, 16)] = one
            zv[pl.ds(16 * j, 16)] = zer
        for kk in range(5):
            idx = s * 5 + kk

            @pl.when(idx < DEG_R // LANE)
            def _():
                pltpu.sync_copy(zv, acc.at[pl.ds(idx * LANE, LANE)])

        plsc.subcore_barrier()
        wid = c * 16 + s
        pltpu.sync_copy(dst_hbm.at[pl.ds(wid * DT, DT)], dst_v)

        def body(j, carry):
            pltpu.async_copy(ones_v, acc.at[dst_v.at[j]], t0, add=True)
            return carry

        lax.fori_loop(0, DT, body, 0)

        def drain(j, carry):
            pltpu.make_async_copy(ones_v, acc.at[dst_v.at[0]], t0).wait()
            return carry

        lax.fori_loop(0, DT, drain, 0)
        plsc.subcore_barrier()
        for kk in range(5):
            blkid = s * 5 + kk

            @pl.when(blkid < DEG_R // LANE)
            def _():
                r0 = blkid * LANE
                pltpu.sync_copy(acc.at[pl.ds(r0, LANE)], zv)
                pltpu.sync_copy(zv, out.at[pl.ds(c * DEG_R + r0, LANE)])

    return k(dst2d)


def _sc_agg(src1, dst2d, ha, hb):
    @functools.partial(
        pl.kernel,
        mesh=_mesh,
        out_type=[
            jax.ShapeDtypeStruct((N, F), jnp.float32),
            jax.ShapeDtypeStruct((N, F), jnp.float32),
        ],
        scratch_types=[
            pltpu.VMEM((16 * LANE,), jnp.int32),
            pltpu.VMEM((16, LANE), jnp.int32),
            pltpu.VMEM((16 * LANE,), jnp.int32),
            pltpu.VMEM((16, LANE), jnp.int32),
            pltpu.VMEM((LANE, F), jnp.float32),
            pltpu.VMEM((LANE, F), jnp.float32),
            pltpu.VMEM_SHARED((ACC_R, F), jnp.float32),
            pltpu.SemaphoreType.DMA,
            pltpu.SemaphoreType.DMA,
            pltpu.SemaphoreType.DMA,
            pltpu.SemaphoreType.DMA,
        ],
    )
    def k(src_hbm, dst_hbm, ha_hbm, hb_hbm, outa, outb,
          sc0, dc0, sc1, dc1, b0, b1, acc, s0, s1, i0, i1):
        c = lax.axis_index("c")
        s = lax.axis_index("s")
        NG = CT // 16

        def run(h_hbm, out_hbm):
            base = s * CT
            NFB = N // LANE
            REM = N - NFB * LANE
            for kk in range(ACC_R // LANE // 16):
                blkid = s * (ACC_R // LANE // 16) + kk
                r0 = blkid * LANE

                @pl.when(blkid < NFB)
                def _():
                    pltpu.sync_copy(h_hbm.at[pl.ds(r0, LANE)], b0)
                    pltpu.sync_copy(b0, acc.at[pl.ds(r0, LANE)])

                @pl.when(blkid == NFB)
                def _():
                    pltpu.sync_copy(h_hbm.at[pl.ds(r0, REM)],
                                    b0.at[pl.ds(0, REM)])
                    pltpu.sync_copy(b0.at[pl.ds(0, REM)],
                                    acc.at[pl.ds(r0, REM)])

            plsc.subcore_barrier()

            def idx_start(g, sc, dc, isem):
                pltpu.async_copy(
                    src_hbm.at[pl.ds(base * LANE + g * 16 * LANE, 16 * LANE)],
                    sc, isem)
                pltpu.async_copy(dst_hbm.at[pl.ds(base + g * 16, 16)], dc, isem)

            def idx_wait(sc, dc, isem):
                pltpu.make_async_copy(
                    src_hbm.at[pl.ds(0, 16 * LANE)], sc, isem).wait()
                pltpu.make_async_copy(
                    dst_hbm.at[pl.ds(base, 16)], dc, isem).wait()

            def do_group(g, sc, dc, isem, scn, dcn, isemn):
                def pair(i, carry):
                    j = 2 * i
                    for t, (bb, ss) in enumerate(((b0, s0), (b1, s1))):
                        jj = j + t
                        pltpu.make_async_copy(
                            h_hbm.at[sc.at[pl.ds(jj * LANE, LANE)]],
                            bb, ss).wait()
                        pltpu.sync_copy(bb, acc.at[dc.at[jj]], add=True)
                        pltpu.async_copy(
                            h_hbm.at[sc.at[pl.ds((jj + 2) * LANE, LANE)]],
                            bb, ss)
                    return carry

                lax.fori_loop(0, 7, pair, 0)
                @pl.when(g + 1 < NG)
                def _():
                    idx_wait(scn, dcn, isemn)

                for t, (bb, ss) in enumerate(((b0, s0), (b1, s1))):
                    jj = 14 + t
                    pltpu.make_async_copy(
                        h_hbm.at[sc.at[pl.ds(jj * LANE, LANE)]], bb, ss).wait()
                    pltpu.sync_copy(bb, acc.at[dc.at[jj]], add=True)

                    @pl.when(g + 1 < NG)
                    def _():
                        pltpu.async_copy(
                            h_hbm.at[scn.at[pl.ds(t * LANE, LANE)]], bb, ss)

                @pl.when(g + 2 < NG)
                def _():
                    idx_start(g + 2, sc, dc, isem)

            idx_start(0, sc0, dc0, i0)
            idx_start(1, sc1, dc1, i1)
            idx_wait(sc0, dc0, i0)
            pltpu.async_copy(h_hbm.at[sc0.at[pl.ds(0, LANE)]], b0, s0)
            pltpu.async_copy(h_hbm.at[sc0.at[pl.ds(LANE, LANE)]], b1, s1)

            def two_groups(p, carry):
                do_group(2 * p, sc0, dc0, i0, sc1, dc1, i1)
                do_group(2 * p + 1, sc1, dc1, i1, sc0, dc0, i0)
                return carry

            lax.fori_loop(0, NG // 2, two_groups, 0)
            plsc.subcore_barrier()
            for kk in range(ACC_R // LANE // 16):
                blkid = s * (ACC_R // LANE // 16) + kk
                r0 = blkid * LANE

                @pl.when(blkid < N // LANE)
                def _():
                    pltpu.sync_copy(acc.at[pl.ds(r0, LANE)], b0)
                    pltpu.sync_copy(b0, out_hbm.at[pl.ds(r0, LANE)])

                @pl.when(blkid == N // LANE)
                def _():
                    rem = N - (N // LANE) * LANE
                    pltpu.sync_copy(acc.at[pl.ds(r0, rem)], b0.at[pl.ds(0, rem)])
                    pltpu.sync_copy(b0.at[pl.ds(0, rem)],
                                    out_hbm.at[pl.ds(r0, rem)])

        @pl.when(c == 0)
        def _():
            run(ha_hbm, outa)

        @pl.when(c == 1)
        def _():
            run(hb_hbm, outb)

    return k(src1, dst2d, ha, hb)


def _tc_a(deg0, deg1, x, W1):
    def body(d0, d1, xr, w, dinv_r, ha_r, hb_r):
        deg = d0[...] + d1[...] + 1.0
        dinv = lax.rsqrt(deg)
        h = jnp.dot(xr[...], w[...], preferred_element_type=jnp.float32) * dinv
        dinv_r[...] = dinv
        ha_r[...] = h[:, :F]
        hb_r[...] = h[:, F:]

    return pl.pallas_call(
        body,
        grid=(N // RB,),
        in_specs=[
            pl.BlockSpec((RB, 1), lambda r: (r, 0)),
            pl.BlockSpec((RB, 1), lambda r: (r, 0)),
            pl.BlockSpec((RB, 128), lambda r: (r, 0)),
            pl.BlockSpec((128, H), lambda r: (0, 0)),
        ],
        out_specs=[
            pl.BlockSpec((RB, 1), lambda r: (r, 0)),
            pl.BlockSpec((RB, F), lambda r: (r, 0)),
            pl.BlockSpec((RB, F), lambda r: (r, 0)),
        ],
        out_shape=[
            jax.ShapeDtypeStruct((N, 1), jnp.float32),
            jax.ShapeDtypeStruct((N, F), jnp.float32),
            jax.ShapeDtypeStruct((N, F), jnp.float32),
        ],
    )(deg0, deg1, x, W1)


def _tc_b(a1a, a1b, dinv, b1r, W2):
    def body(aa, ab, dv, br, w, h1a_r, h1b_r, ga_r, gb_r):
        agg = jnp.concatenate([aa[...], ab[...]], axis=1)
        dinv_b = dv[...]
        h1 = jnp.maximum(dinv_b * agg + br[...], 0.0)
        g = jnp.dot(h1, w[...], preferred_element_type=jnp.float32) * dinv_b
        h1a_r[...] = h1[:, :F]
        h1b_r[...] = h1[:, F:]
        ga_r[...] = g[:, :F]
        gb_r[...] = g[:, F:]

    return pl.pallas_call(
        body,
        grid=(N // RB,),
        in_specs=[
            pl.BlockSpec((RB, F), lambda r: (r, 0)),
            pl.BlockSpec((RB, F), lambda r: (r, 0)),
            pl.BlockSpec((RB, 1), lambda r: (r, 0)),
            pl.BlockSpec((1, H), lambda r: (0, 0)),
            pl.BlockSpec((H, H), lambda r: (0, 0)),
        ],
        out_specs=[
            pl.BlockSpec((RB, F), lambda r: (r, 0)),
            pl.BlockSpec((RB, F), lambda r: (r, 0)),
            pl.BlockSpec((RB, F), lambda r: (r, 0)),
            pl.BlockSpec((RB, F), lambda r: (r, 0)),
        ],
        out_shape=[
            jax.ShapeDtypeStruct((N, F), jnp.float32),
            jax.ShapeDtypeStruct((N, F), jnp.float32),
            jax.ShapeDtypeStruct((N, F), jnp.float32),
            jax.ShapeDtypeStruct((N, F), jnp.float32),
        ],
    )(a1a, a1b, dinv, b1r, W2)


def _tc_c(a2a, a2b, h1a, h1b, dinv, b2r, gam, bet):
    def body(aa, ab, ha, hb, dv, br, gm, bt, out_r):
        conv = jnp.concatenate([aa[...], ab[...]], axis=1)
        dinv_b = dv[...]
        h2 = (jnp.concatenate([ha[...], hb[...]], axis=1)
              + jnp.maximum(dinv_b * conv + br[...], 0.0))
        mu = jnp.mean(h2, axis=1, keepdims=True)
        d = h2 - mu
        var = jnp.mean(d * d, axis=1, keepdims=True)
        out_r[...] = gm[...] * d * lax.rsqrt(var + 1e-5) + bt[...]

    return pl.pallas_call(
        body,
        grid=(N // RB,),
        in_specs=[
            pl.BlockSpec((RB, F), lambda r: (r, 0)),
            pl.BlockSpec((RB, F), lambda r: (r, 0)),
            pl.BlockSpec((RB, F), lambda r: (r, 0)),
            pl.BlockSpec((RB, F), lambda r: (r, 0)),
            pl.BlockSpec((RB, 1), lambda r: (r, 0)),
            pl.BlockSpec((1, H), lambda r: (0, 0)),
            pl.BlockSpec((1, H), lambda r: (0, 0)),
            pl.BlockSpec((1, H), lambda r: (0, 0)),
        ],
        out_specs=pl.BlockSpec((RB, H), lambda r: (r, 0)),
        out_shape=jax.ShapeDtypeStruct((N, H), jnp.float32),
    )(a2a, a2b, h1a, h1b, dinv, b2r, gam, bet)


def kernel(x, edge_index, W1, b1, W2, b2, gamma, beta):
    ei = edge_index.astype(jnp.int32)
    src, dst = ei[0], ei[1]
    pad = PE - E
    pad_src = jnp.arange(pad, dtype=jnp.int32) % 16
    pad_dst = N + jnp.arange(pad, dtype=jnp.int32) % 16
    src1 = jnp.concatenate([src, pad_src])
    dst2d = jnp.concatenate([dst, pad_dst]).reshape(ROWS2D, LANE)

    degp = _sc_deg(dst2d)
    deg0 = degp[:N].reshape(N, 1)
    deg1 = degp[DEG_R:DEG_R + N].reshape(N, 1)

    dinv, hs1a, hs1b = _tc_a(deg0, deg1, x, W1)
    a1a, a1b = _sc_agg(src1, dst2d, hs1a, hs1b)
    h1a, h1b, g2a, g2b = _tc_b(a1a, a1b, dinv, b1.reshape(1, H), W2)
    a2a, a2b = _sc_agg(src1, dst2d, g2a, g2b)
    return _tc_c(a2a, a2b, h1a, h1b, dinv,
                 b2.reshape(1, H), gamma.reshape(1, H), beta.reshape(1, H))


# speedup vs baseline: 1.0678x; 1.0678x over previous
import functools

import jax
import jax.numpy as jnp
from jax import lax
from jax.experimental import pallas as pl
from jax.experimental.pallas import tpu as pltpu
from jax.experimental.pallas import tpu_sc as plsc

N = 10000
E = 320000
LANE = 128
CT = 160
PE = 16 * CT * LANE
ROWS2D = PE // LANE
DT = ROWS2D // 32
ACC_R = 10240
DEG_R = 10112
F = 128
H = 256
RB = 1000

_mesh = plsc.VectorSubcoreMesh(core_axis_name="c", subcore_axis_name="s")


def _sc_deg(e3d):
    @functools.partial(
        pl.kernel,
        mesh=_mesh,
        out_type=jax.ShapeDtypeStruct((2 * DEG_R,), jnp.float32),
        scratch_types=[
            pltpu.VMEM((DT, LANE), jnp.int32),
            pltpu.VMEM((LANE,), jnp.float32),
            pltpu.VMEM((LANE,), jnp.float32),
            pltpu.VMEM_SHARED((DEG_R,), jnp.float32),
            pltpu.SemaphoreType.DMA,
        ],
    )
    def k(e_hbm, out, dst_v, ones_v, zv, acc, t0):
        c = lax.axis_index("c")
        s = lax.axis_index("s")
        one = jnp.full((16,), 1.0, dtype=jnp.float32)
        zer = jnp.zeros((16,), dtype=jnp.float32)
        for j in range(LANE // 16):
            ones_v[pl.ds(16 * j, 16)] = one
            zv[pl.ds(16 * j, 16)] = zer
        for kk in range(5):
            idx = s * 5 + kk

            @pl.when(idx < DEG_R // LANE)
            def _():
                pltpu.sync_copy(zv, acc.at[pl.ds(idx * LANE, LANE)])

        plsc.subcore_barrier()
        wid = c * 16 + s
        pltpu.sync_copy(e_hbm.at[1, pl.ds(wid * DT, DT)], dst_v)

        def body(j, carry):
            pltpu.async_copy(ones_v, acc.at[dst_v.at[j]], t0, add=True)
            return carry

        lax.fori_loop(0, DT, body, 0)

        def drain(j, carry):
            pltpu.make_async_copy(ones_v, acc.at[dst_v.at[0]], t0).wait()
            return carry

        lax.fori_loop(0, DT, drain, 0)
        plsc.subcore_barrier()
        for kk in range(5):
            blkid = s * 5 + kk

            @pl.when(blkid < DEG_R // LANE)
            def _():
                r0 = blkid * LANE
                pltpu.sync_copy(acc.at[pl.ds(r0, LANE)], zv)
                pltpu.sync_copy(zv, out.at[pl.ds(c * DEG_R + r0, LANE)])

    return k(e3d)


def _sc_agg(e3d, ha, hb):
    @functools.partial(
        pl.kernel,
        mesh=_mesh,
        out_type=[
            jax.ShapeDtypeStruct((N, F), jnp.float32),
            jax.ShapeDtypeStruct((N, F), jnp.float32),
        ],
        scratch_types=[
            pltpu.VMEM((16, LANE), jnp.int32),
            pltpu.VMEM((16, LANE), jnp.int32),
            pltpu.VMEM((16, LANE), jnp.int32),
            pltpu.VMEM((16, LANE), jnp.int32),
            pltpu.VMEM((LANE, F), jnp.float32),
            pltpu.VMEM((LANE, F), jnp.float32),
            pltpu.VMEM_SHARED((ACC_R, F), jnp.float32),
            pltpu.SemaphoreType.DMA,
            pltpu.SemaphoreType.DMA,
            pltpu.SemaphoreType.DMA,
            pltpu.SemaphoreType.DMA,
        ],
    )
    def k(e_hbm, ha_hbm, hb_hbm, outa, outb,
          sc0, dc0, sc1, dc1, b0, b1, acc, s0, s1, i0, i1):
        c = lax.axis_index("c")
        s = lax.axis_index("s")
        NG = CT // 16

        def run(h_hbm, out_hbm):
            base = s * CT
            NFB = N // LANE
            REM = N - NFB * LANE
            for kk in range(ACC_R // LANE // 16):
                blkid = s * (ACC_R // LANE // 16) + kk
                r0 = blkid * LANE

                @pl.when(blkid < NFB)
                def _():
                    pltpu.sync_copy(h_hbm.at[pl.ds(r0, LANE)], b0)
                    pltpu.sync_copy(b0, acc.at[pl.ds(r0, LANE)])

                @pl.when(blkid == NFB)
                def _():
                    pltpu.sync_copy(h_hbm.at[pl.ds(r0, REM)],
                                    b0.at[pl.ds(0, REM)])
                    pltpu.sync_copy(b0.at[pl.ds(0, REM)],
                                    acc.at[pl.ds(r0, REM)])

            plsc.subcore_barrier()

            def idx_start(g, sc, dc, isem):
                pltpu.async_copy(
                    e_hbm.at[0, pl.ds(base + g * 16, 16)], sc, isem)
                pltpu.async_copy(
                    e_hbm.at[1, pl.ds(base + g * 16, 16)], dc, isem)

            def idx_wait(sc, dc, isem):
                pltpu.make_async_copy(
                    e_hbm.at[0, pl.ds(base, 16)], sc, isem).wait()
                pltpu.make_async_copy(
                    e_hbm.at[1, pl.ds(base, 16)], dc, isem).wait()

            def do_group(g, sc, dc, isem, scn, dcn, isemn):
                def pair(i, carry):
                    j = 2 * i
                    for t, (bb, ss) in enumerate(((b0, s0), (b1, s1))):
                        jj = j + t
                        pltpu.make_async_copy(
                            h_hbm.at[sc.at[jj]], bb, ss).wait()
                        pltpu.sync_copy(bb, acc.at[dc.at[jj]], add=True)
                        pltpu.async_copy(h_hbm.at[sc.at[jj + 2]], bb, ss)
                    return carry

                lax.fori_loop(0, 7, pair, 0)
                @pl.when(g + 1 < NG)
                def _():
                    idx_wait(scn, dcn, isemn)

                for t, (bb, ss) in enumerate(((b0, s0), (b1, s1))):
                    jj = 14 + t
                    pltpu.make_async_copy(h_hbm.at[sc.at[jj]], bb, ss).wait()
                    pltpu.sync_copy(bb, acc.at[dc.at[jj]], add=True)

                    @pl.when(g + 1 < NG)
                    def _():
                        pltpu.async_copy(h_hbm.at[scn.at[t]], bb, ss)

                @pl.when(g + 2 < NG)
                def _():
                    idx_start(g + 2, sc, dc, isem)

            idx_start(0, sc0, dc0, i0)
            idx_start(1, sc1, dc1, i1)
            idx_wait(sc0, dc0, i0)
            pltpu.async_copy(h_hbm.at[sc0.at[0]], b0, s0)
            pltpu.async_copy(h_hbm.at[sc0.at[1]], b1, s1)

            def two_groups(p, carry):
                do_group(2 * p, sc0, dc0, i0, sc1, dc1, i1)
                do_group(2 * p + 1, sc1, dc1, i1, sc0, dc0, i0)
                return carry

            lax.fori_loop(0, NG // 2, two_groups, 0)
            plsc.subcore_barrier()
            for kk in range(ACC_R // LANE // 16):
                blkid = s * (ACC_R // LANE // 16) + kk
                r0 = blkid * LANE

                @pl.when(blkid < N // LANE)
                def _():
                    pltpu.sync_copy(acc.at[pl.ds(r0, LANE)], b0)
                    pltpu.sync_copy(b0, out_hbm.at[pl.ds(r0, LANE)])

                @pl.when(blkid == N // LANE)
                def _():
                    rem = N - (N // LANE) * LANE
                    pltpu.sync_copy(acc.at[pl.ds(r0, rem)], b0.at[pl.ds(0, rem)])
                    pltpu.sync_copy(b0.at[pl.ds(0, rem)],
                                    out_hbm.at[pl.ds(r0, rem)])

        @pl.when(c == 0)
        def _():
            run(ha_hbm, outa)

        @pl.when(c == 1)
        def _():
            run(hb_hbm, outb)

    return k(e3d, ha, hb)


def _tc_a(deg0, deg1, x, W1):
    def body(d0, d1, xr, w, dinv_r, ha_r, hb_r):
        deg = d0[...] + d1[...] + 1.0
        dinv = lax.rsqrt(deg)
        h = jnp.dot(xr[...], w[...], preferred_element_type=jnp.float32) * dinv
        dinv_r[...] = dinv
        ha_r[...] = h[:, :F]
        hb_r[...] = h[:, F:]

    return pl.pallas_call(
        body,
        grid=(N // RB,),
        in_specs=[
            pl.BlockSpec((RB, 1), lambda r: (r, 0)),
            pl.BlockSpec((RB, 1), lambda r: (r, 0)),
            pl.BlockSpec((RB, 128), lambda r: (r, 0)),
            pl.BlockSpec((128, H), lambda r: (0, 0)),
        ],
        out_specs=[
            pl.BlockSpec((RB, 1), lambda r: (r, 0)),
            pl.BlockSpec((RB, F), lambda r: (r, 0)),
            pl.BlockSpec((RB, F), lambda r: (r, 0)),
        ],
        out_shape=[
            jax.ShapeDtypeStruct((N, 1), jnp.float32),
            jax.ShapeDtypeStruct((N, F), jnp.float32),
            jax.ShapeDtypeStruct((N, F), jnp.float32),
        ],
    )(deg0, deg1, x, W1)


def _tc_b(a1a, a1b, dinv, b1r, W2):
    def body(aa, ab, dv, br, w, h1a_r, h1b_r, ga_r, gb_r):
        agg = jnp.concatenate([aa[...], ab[...]], axis=1)
        dinv_b = dv[...]
        h1 = jnp.maximum(dinv_b * agg + br[...], 0.0)
        g = jnp.dot(h1, w[...], preferred_element_type=jnp.float32) * dinv_b
        h1a_r[...] = h1[:, :F]
        h1b_r[...] = h1[:, F:]
        ga_r[...] = g[:, :F]
        gb_r[...] = g[:, F:]

    return pl.pallas_call(
        body,
        grid=(N // RB,),
        in_specs=[
            pl.BlockSpec((RB, F), lambda r: (r, 0)),
            pl.BlockSpec((RB, F), lambda r: (r, 0)),
            pl.BlockSpec((RB, 1), lambda r: (r, 0)),
            pl.BlockSpec((1, H), lambda r: (0, 0)),
            pl.BlockSpec((H, H), lambda r: (0, 0)),
        ],
        out_specs=[
            pl.BlockSpec((RB, F), lambda r: (r, 0)),
            pl.BlockSpec((RB, F), lambda r: (r, 0)),
            pl.BlockSpec((RB, F), lambda r: (r, 0)),
            pl.BlockSpec((RB, F), lambda r: (r, 0)),
        ],
        out_shape=[
            jax.ShapeDtypeStruct((N, F), jnp.float32),
            jax.ShapeDtypeStruct((N, F), jnp.float32),
            jax.ShapeDtypeStruct((N, F), jnp.float32),
            jax.ShapeDtypeStruct((N, F), jnp.float32),
        ],
    )(a1a, a1b, dinv, b1r, W2)


def _tc_c(a2a, a2b, h1a, h1b, dinv, b2r, gam, bet):
    def body(aa, ab, ha, hb, dv, br, gm, bt, out_r):
        conv = jnp.concatenate([aa[...], ab[...]], axis=1)
        dinv_b = dv[...]
        h2 = (jnp.concatenate([ha[...], hb[...]], axis=1)
              + jnp.maximum(dinv_b * conv + br[...], 0.0))
        mu = jnp.mean(h2, axis=1, keepdims=True)
        d = h2 - mu
        var = jnp.mean(d * d, axis=1, keepdims=True)
        out_r[...] = gm[...] * d * lax.rsqrt(var + 1e-5) + bt[...]

    return pl.pallas_call(
        body,
        grid=(N // RB,),
        in_specs=[
            pl.BlockSpec((RB, F), lambda r: (r, 0)),
            pl.BlockSpec((RB, F), lambda r: (r, 0)),
            pl.BlockSpec((RB, F), lambda r: (r, 0)),
            pl.BlockSpec((RB, F), lambda r: (r, 0)),
            pl.BlockSpec((RB, 1), lambda r: (r, 0)),
            pl.BlockSpec((1, H), lambda r: (0, 0)),
            pl.BlockSpec((1, H), lambda r: (0, 0)),
            pl.BlockSpec((1, H), lambda r: (0, 0)),
        ],
        out_specs=pl.BlockSpec((RB, H), lambda r: (r, 0)),
        out_shape=jax.ShapeDtypeStruct((N, H), jnp.float32),
    )(a2a, a2b, h1a, h1b, dinv, b2r, gam, bet)


def kernel(x, edge_index, W1, b1, W2, b2, gamma, beta):
    ei = edge_index.astype(jnp.int32)
    pad = PE - E
    pad_src = jnp.arange(pad, dtype=jnp.int32) % 16
    pad_dst = N + jnp.arange(pad, dtype=jnp.int32) % 16
    e3d = jnp.concatenate(
        [ei, jnp.stack([pad_src, pad_dst])], axis=1).reshape(2, ROWS2D, LANE)

    degp = _sc_deg(e3d)
    deg0 = degp[:N].reshape(N, 1)
    deg1 = degp[DEG_R:DEG_R + N].reshape(N, 1)

    dinv, hs1a, hs1b = _tc_a(deg0, deg1, x, W1)
    a1a, a1b = _sc_agg(e3d, hs1a, hs1b)
    h1a, h1b, g2a, g2b = _tc_b(a1a, a1b, dinv, b1.reshape(1, H), W2)
    a2a, a2b = _sc_agg(e3d, g2a, g2b)
    return _tc_c(a2a, a2b, h1a, h1b, dinv,
                 b2.reshape(1, H), gamma.reshape(1, H), beta.reshape(1, H))


# speedup vs baseline: 1.0932x; 1.0238x over previous
import functools

import jax
import jax.numpy as jnp
from jax import lax
from jax.experimental import pallas as pl
from jax.experimental.pallas import tpu as pltpu
from jax.experimental.pallas import tpu_sc as plsc

N = 10000
E = 320000
LANE = 128
CT = 160
PE = 16 * CT * LANE
ROWS2D = PE // LANE
DT = ROWS2D // 32
ACC_R = 10240
DEG_R = 10112
F = 128
H = 256
RB = 2000

_mesh = plsc.VectorSubcoreMesh(core_axis_name="c", subcore_axis_name="s")


def _sc_deg(e3d):
    @functools.partial(
        pl.kernel,
        mesh=_mesh,
        out_type=jax.ShapeDtypeStruct((2 * DEG_R,), jnp.float32),
        scratch_types=[
            pltpu.VMEM((DT, LANE), jnp.int32),
            pltpu.VMEM((LANE,), jnp.float32),
            pltpu.VMEM((LANE,), jnp.float32),
            pltpu.VMEM_SHARED((DEG_R,), jnp.float32),
            pltpu.SemaphoreType.DMA,
        ],
    )
    def k(e_hbm, out, dst_v, ones_v, zv, acc, t0):
        c = lax.axis_index("c")
        s = lax.axis_index("s")
        one = jnp.full((16,), 1.0, dtype=jnp.float32)
        zer = jnp.zeros((16,), dtype=jnp.float32)
        for j in range(LANE // 16):
            ones_v[pl.ds(16 * j, 16)] = one
            zv[pl.ds(16 * j, 16)] = zer
        for kk in range(5):
            idx = s * 5 + kk

            @pl.when(idx < DEG_R // LANE)
            def _():
                pltpu.sync_copy(zv, acc.at[pl.ds(idx * LANE, LANE)])

        plsc.subcore_barrier()
        wid = c * 16 + s
        pltpu.sync_copy(e_hbm.at[1, pl.ds(wid * DT, DT)], dst_v)

        def body(j, carry):
            pltpu.async_copy(ones_v, acc.at[dst_v.at[j]], t0, add=True)
            return carry

        lax.fori_loop(0, DT, body, 0)

        def drain(j, carry):
            pltpu.make_async_copy(ones_v, acc.at[dst_v.at[0]], t0).wait()
            return carry

        lax.fori_loop(0, DT, drain, 0)
        plsc.subcore_barrier()
        for kk in range(5):
            blkid = s * 5 + kk

            @pl.when(blkid < DEG_R // LANE)
            def _():
                r0 = blkid * LANE
                pltpu.sync_copy(acc.at[pl.ds(r0, LANE)], zv)
                pltpu.sync_copy(zv, out.at[pl.ds(c * DEG_R + r0, LANE)])

    return k(e3d)


def _sc_agg(e3d, ha, hb):
    @functools.partial(
        pl.kernel,
        mesh=_mesh,
        out_type=[
            jax.ShapeDtypeStruct((N, F), jnp.float32),
            jax.ShapeDtypeStruct((N, F), jnp.float32),
        ],
        scratch_types=[
            pltpu.VMEM((16, LANE), jnp.int32),
            pltpu.VMEM((16, LANE), jnp.int32),
            pltpu.VMEM((16, LANE), jnp.int32),
            pltpu.VMEM((16, LANE), jnp.int32),
            pltpu.VMEM((LANE, F), jnp.float32),
            pltpu.VMEM((LANE, F), jnp.float32),
            pltpu.VMEM_SHARED((ACC_R, F), jnp.float32),
            pltpu.SemaphoreType.DMA,
            pltpu.SemaphoreType.DMA,
            pltpu.SemaphoreType.DMA,
            pltpu.SemaphoreType.DMA,
        ],
    )
    def k(e_hbm, ha_hbm, hb_hbm, outa, outb,
          sc0, dc0, sc1, dc1, b0, b1, acc, s0, s1, i0, i1):
        c = lax.axis_index("c")
        s = lax.axis_index("s")
        NG = CT // 16

        def run(h_hbm, out_hbm):
            base = s * CT
            NFB = N // LANE
            REM = N - NFB * LANE
            for kk in range(ACC_R // LANE // 16):
                blkid = s * (ACC_R // LANE // 16) + kk
                r0 = blkid * LANE

                @pl.when(blkid < NFB)
                def _():
                    pltpu.sync_copy(h_hbm.at[pl.ds(r0, LANE)], b0)
                    pltpu.sync_copy(b0, acc.at[pl.ds(r0, LANE)])

                @pl.when(blkid == NFB)
                def _():
                    pltpu.sync_copy(h_hbm.at[pl.ds(r0, REM)],
                                    b0.at[pl.ds(0, REM)])
                    pltpu.sync_copy(b0.at[pl.ds(0, REM)],
                                    acc.at[pl.ds(r0, REM)])

            plsc.subcore_barrier()

            def idx_start(g, sc, dc, isem):
                pltpu.async_copy(
                    e_hbm.at[0, pl.ds(base + g * 16, 16)], sc, isem)
                pltpu.async_copy(
                    e_hbm.at[1, pl.ds(base + g * 16, 16)], dc, isem)

            def idx_wait(sc, dc, isem):
                pltpu.make_async_copy(
                    e_hbm.at[0, pl.ds(base, 16)], sc, isem).wait()
                pltpu.make_async_copy(
                    e_hbm.at[1, pl.ds(base, 16)], dc, isem).wait()

            def do_group(g, sc, dc, isem, scn, dcn, isemn):
                def pair(i, carry):
                    j = 2 * i
                    for t, (bb, ss) in enumerate(((b0, s0), (b1, s1))):
                        jj = j + t
                        pltpu.make_async_copy(
                            h_hbm.at[sc.at[jj]], bb, ss).wait()
                        pltpu.sync_copy(bb, acc.at[dc.at[jj]], add=True)
                        pltpu.async_copy(h_hbm.at[sc.at[jj + 2]], bb, ss)
                    return carry

                lax.fori_loop(0, 7, pair, 0)
                @pl.when(g + 1 < NG)
                def _():
                    idx_wait(scn, dcn, isemn)

                for t, (bb, ss) in enumerate(((b0, s0), (b1, s1))):
                    jj = 14 + t
                    pltpu.make_async_copy(h_hbm.at[sc.at[jj]], bb, ss).wait()
                    pltpu.sync_copy(bb, acc.at[dc.at[jj]], add=True)

                    @pl.when(g + 1 < NG)
                    def _():
                        pltpu.async_copy(h_hbm.at[scn.at[t]], bb, ss)

                @pl.when(g + 2 < NG)
                def _():
                    idx_start(g + 2, sc, dc, isem)

            idx_start(0, sc0, dc0, i0)
            idx_start(1, sc1, dc1, i1)
            idx_wait(sc0, dc0, i0)
            pltpu.async_copy(h_hbm.at[sc0.at[0]], b0, s0)
            pltpu.async_copy(h_hbm.at[sc0.at[1]], b1, s1)

            def two_groups(p, carry):
                do_group(2 * p, sc0, dc0, i0, sc1, dc1, i1)
                do_group(2 * p + 1, sc1, dc1, i1, sc0, dc0, i0)
                return carry

            lax.fori_loop(0, NG // 2, two_groups, 0)
            plsc.subcore_barrier()
            for kk in range(ACC_R // LANE // 16):
                blkid = s * (ACC_R // LANE // 16) + kk
                r0 = blkid * LANE

                @pl.when(blkid < N // LANE)
                def _():
                    pltpu.sync_copy(acc.at[pl.ds(r0, LANE)], b0)
                    pltpu.sync_copy(b0, out_hbm.at[pl.ds(r0, LANE)])

                @pl.when(blkid == N // LANE)
                def _():
                    rem = N - (N // LANE) * LANE
                    pltpu.sync_copy(acc.at[pl.ds(r0, rem)], b0.at[pl.ds(0, rem)])
                    pltpu.sync_copy(b0.at[pl.ds(0, rem)],
                                    out_hbm.at[pl.ds(r0, rem)])

        @pl.when(c == 0)
        def _():
            run(ha_hbm, outa)

        @pl.when(c == 1)
        def _():
            run(hb_hbm, outb)

    return k(e3d, ha, hb)


def _tc_a(degT, x, W1):
    def body(dg, xr, w, dinv_r, ha_r, hb_r):
        deg = dg[:, 0:1] + dg[:, 1:2] + 1.0
        dinv = lax.rsqrt(deg)
        h = jnp.dot(xr[...], w[...], preferred_element_type=jnp.float32) * dinv
        dinv_r[...] = dinv
        ha_r[...] = h[:, :F]
        hb_r[...] = h[:, F:]

    return pl.pallas_call(
        body,
        grid=(N // RB,),
        in_specs=[
            pl.BlockSpec((RB, 2), lambda r: (r, 0)),
            pl.BlockSpec((RB, 128), lambda r: (r, 0)),
            pl.BlockSpec((128, H), lambda r: (0, 0)),
        ],
        out_specs=[
            pl.BlockSpec((RB, 1), lambda r: (r, 0)),
            pl.BlockSpec((RB, F), lambda r: (r, 0)),
            pl.BlockSpec((RB, F), lambda r: (r, 0)),
        ],
        out_shape=[
            jax.ShapeDtypeStruct((N, 1), jnp.float32),
            jax.ShapeDtypeStruct((N, F), jnp.float32),
            jax.ShapeDtypeStruct((N, F), jnp.float32),
        ],
    )(degT, x, W1)


def _tc_b(a1a, a1b, dinv, b1r, W2):
    def body(aa, ab, dv, br, w, h1a_r, h1b_r, ga_r, gb_r):
        agg = jnp.concatenate([aa[...], ab[...]], axis=1)
        dinv_b = dv[...]
        h1 = jnp.maximum(dinv_b * agg + br[...], 0.0)
        g = jnp.dot(h1, w[...], preferred_element_type=jnp.float32) * dinv_b
        h1a_r[...] = h1[:, :F]
        h1b_r[...] = h1[:, F:]
        ga_r[...] = g[:, :F]
        gb_r[...] = g[:, F:]

    return pl.pallas_call(
        body,
        grid=(N // RB,),
        in_specs=[
            pl.BlockSpec((RB, F), lambda r: (r, 0)),
            pl.BlockSpec((RB, F), lambda r: (r, 0)),
            pl.BlockSpec((RB, 1), lambda r: (r, 0)),
            pl.BlockSpec((1, H), lambda r: (0, 0)),
            pl.BlockSpec((H, H), lambda r: (0, 0)),
        ],
        out_specs=[
            pl.BlockSpec((RB, F), lambda r: (r, 0)),
            pl.BlockSpec((RB, F), lambda r: (r, 0)),
            pl.BlockSpec((RB, F), lambda r: (r, 0)),
            pl.BlockSpec((RB, F), lambda r: (r, 0)),
        ],
        out_shape=[
            jax.ShapeDtypeStruct((N, F), jnp.float32),
            jax.ShapeDtypeStruct((N, F), jnp.float32),
            jax.ShapeDtypeStruct((N, F), jnp.float32),
            jax.ShapeDtypeStruct((N, F), jnp.float32),
        ],
    )(a1a, a1b, dinv, b1r, W2)


def _tc_c(a2a, a2b, h1a, h1b, dinv, b2r, gam, bet):
    def body(aa, ab, ha, hb, dv, br, gm, bt, out_r):
        conv = jnp.concatenate([aa[...], ab[...]], axis=1)
        dinv_b = dv[...]
        h2 = (jnp.concatenate([ha[...], hb[...]], axis=1)
              + jnp.maximum(dinv_b * conv + br[...], 0.0))
        mu = jnp.mean(h2, axis=1, keepdims=True)
        d = h2 - mu
        var = jnp.mean(d * d, axis=1, keepdims=True)
        out_r[...] = gm[...] * d * lax.rsqrt(var + 1e-5) + bt[...]

    return pl.pallas_call(
        body,
        grid=(N // RB,),
        in_specs=[
            pl.BlockSpec((RB, F), lambda r: (r, 0)),
            pl.BlockSpec((RB, F), lambda r: (r, 0)),
            pl.BlockSpec((RB, F), lambda r: (r, 0)),
            pl.BlockSpec((RB, F), lambda r: (r, 0)),
            pl.BlockSpec((RB, 1), lambda r: (r, 0)),
            pl.BlockSpec((1, H), lambda r: (0, 0)),
            pl.BlockSpec((1, H), lambda r: (0, 0)),
            pl.BlockSpec((1, H), lambda r: (0, 0)),
        ],
        out_specs=pl.BlockSpec((RB, H), lambda r: (r, 0)),
        out_shape=jax.ShapeDtypeStruct((N, H), jnp.float32),
    )(a2a, a2b, h1a, h1b, dinv, b2r, gam, bet)


def kernel(x, edge_index, W1, b1, W2, b2, gamma, beta):
    ei = edge_index.astype(jnp.int32)
    pad = PE - E
    pad_src = jnp.arange(pad, dtype=jnp.int32) % 16
    pad_dst = N + jnp.arange(pad, dtype=jnp.int32) % 16
    e3d = jnp.concatenate(
        [ei, jnp.stack([pad_src, pad_dst])], axis=1).reshape(2, ROWS2D, LANE)

    degp = _sc_deg(e3d)
    degT = degp.reshape(2, DEG_R).T[:N]

    dinv, hs1a, hs1b = _tc_a(degT, x, W1)
    a1a, a1b = _sc_agg(e3d, hs1a, hs1b)
    h1a, h1b, g2a, g2b = _tc_b(a1a, a1b, dinv, b1.reshape(1, H), W2)
    a2a, a2b = _sc_agg(e3d, g2a, g2b)
    return _tc_c(a2a, a2b, h1a, h1b, dinv,
                 b2.reshape(1, H), gamma.reshape(1, H), beta.reshape(1, H))


# speedup vs baseline: 1.0968x; 1.0032x over previous
import functools

import jax
import jax.numpy as jnp
from jax import lax
from jax.experimental import pallas as pl
from jax.experimental.pallas import tpu as pltpu
from jax.experimental.pallas import tpu_sc as plsc

N = 10000
E = 320000
LANE = 128
CT = 160
PE = 16 * CT * LANE
ROWS2D = PE // LANE
DT = ROWS2D // 32
ACC_R = 10240
DEG_R = 10112
F = 128
H = 256
RB = 2000

_mesh = plsc.VectorSubcoreMesh(core_axis_name="c", subcore_axis_name="s")


def _sc_deg(e3d):
    @functools.partial(
        pl.kernel,
        mesh=_mesh,
        out_type=jax.ShapeDtypeStruct((2 * DEG_R,), jnp.float32),
        scratch_types=[
            pltpu.VMEM((DT, LANE), jnp.int32),
            pltpu.VMEM((LANE,), jnp.float32),
            pltpu.VMEM((LANE,), jnp.float32),
            pltpu.VMEM_SHARED((DEG_R,), jnp.float32),
            pltpu.SemaphoreType.DMA,
        ],
    )
    def k(e_hbm, out, dst_v, ones_v, zv, acc, t0):
        c = lax.axis_index("c")
        s = lax.axis_index("s")
        one = jnp.full((16,), 1.0, dtype=jnp.float32)
        zer = jnp.zeros((16,), dtype=jnp.float32)
        for j in range(LANE // 16):
            ones_v[pl.ds(16 * j, 16)] = one
            zv[pl.ds(16 * j, 16)] = zer
        for kk in range(5):
            idx = s * 5 + kk

            @pl.when(idx < DEG_R // LANE)
            def _():
                pltpu.sync_copy(zv, acc.at[pl.ds(idx * LANE, LANE)])

        plsc.subcore_barrier()
        wid = c * 16 + s
        pltpu.sync_copy(e_hbm.at[1, pl.ds(wid * DT, DT)], dst_v)

        def body(j, carry):
            pltpu.async_copy(ones_v, acc.at[dst_v.at[j]], t0, add=True)
            return carry

        lax.fori_loop(0, DT, body, 0)

        def drain(j, carry):
            pltpu.make_async_copy(ones_v, acc.at[dst_v.at[0]], t0).wait()
            return carry

        lax.fori_loop(0, DT, drain, 0)
        plsc.subcore_barrier()
        for kk in range(5):
            blkid = s * 5 + kk

            @pl.when(blkid < DEG_R // LANE)
            def _():
                r0 = blkid * LANE
                pltpu.sync_copy(acc.at[pl.ds(r0, LANE)], zv)
                pltpu.sync_copy(zv, out.at[pl.ds(c * DEG_R + r0, LANE)])

    return k(e3d)


def _sc_agg(e3d, ha, hb):
    @functools.partial(
        pl.kernel,
        mesh=_mesh,
        out_type=[
            jax.ShapeDtypeStruct((N, F), jnp.float32),
            jax.ShapeDtypeStruct((N, F), jnp.float32),
        ],
        scratch_types=[
            pltpu.VMEM((16, LANE), jnp.int32),
            pltpu.VMEM((16, LANE), jnp.int32),
            pltpu.VMEM((16, LANE), jnp.int32),
            pltpu.VMEM((16, LANE), jnp.int32),
            pltpu.VMEM((LANE, F), jnp.float32),
            pltpu.VMEM((LANE, F), jnp.float32),
            pltpu.VMEM_SHARED((ACC_R, F), jnp.float32),
            pltpu.SemaphoreType.DMA,
            pltpu.SemaphoreType.DMA,
            pltpu.SemaphoreType.DMA,
            pltpu.SemaphoreType.DMA,
        ],
    )
    def k(e_hbm, ha_hbm, hb_hbm, outa, outb,
          sc0, dc0, sc1, dc1, b0, b1, acc, s0, s1, i0, i1):
        c = lax.axis_index("c")
        s = lax.axis_index("s")
        NG = CT // 16

        def run(h_hbm, out_hbm):
            base = s * CT

            def idx_start(g, sc, dc, isem):
                pltpu.async_copy(
                    e_hbm.at[0, pl.ds(base + g * 16, 16)], sc, isem)
                pltpu.async_copy(
                    e_hbm.at[1, pl.ds(base + g * 16, 16)], dc, isem)

            def idx_wait(sc, dc, isem):
                pltpu.make_async_copy(
                    e_hbm.at[0, pl.ds(base, 16)], sc, isem).wait()
                pltpu.make_async_copy(
                    e_hbm.at[1, pl.ds(base, 16)], dc, isem).wait()

            idx_start(0, sc0, dc0, i0)
            idx_start(1, sc1, dc1, i1)

            NFB = N // LANE
            REM = N - NFB * LANE
            for kk in range(ACC_R // LANE // 16):
                blkid = s * (ACC_R // LANE // 16) + kk
                r0 = blkid * LANE

                @pl.when(blkid < NFB)
                def _():
                    pltpu.sync_copy(h_hbm.at[pl.ds(r0, LANE)], b0)
                    pltpu.sync_copy(b0, acc.at[pl.ds(r0, LANE)])

                @pl.when(blkid == NFB)
                def _():
                    pltpu.sync_copy(h_hbm.at[pl.ds(r0, REM)],
                                    b0.at[pl.ds(0, REM)])
                    pltpu.sync_copy(b0.at[pl.ds(0, REM)],
                                    acc.at[pl.ds(r0, REM)])

            idx_wait(sc0, dc0, i0)
            pltpu.async_copy(h_hbm.at[sc0.at[0]], b0, s0)
            pltpu.async_copy(h_hbm.at[sc0.at[1]], b1, s1)
            plsc.subcore_barrier()

            def do_group(g, sc, dc, isem, scn, dcn, isemn):
                def pair(i, carry):
                    j = 2 * i
                    for t, (bb, ss) in enumerate(((b0, s0), (b1, s1))):
                        jj = j + t
                        pltpu.make_async_copy(
                            h_hbm.at[sc.at[jj]], bb, ss).wait()
                        pltpu.sync_copy(bb, acc.at[dc.at[jj]], add=True)
                        pltpu.async_copy(h_hbm.at[sc.at[jj + 2]], bb, ss)
                    return carry

                lax.fori_loop(0, 7, pair, 0)
                @pl.when(g + 1 < NG)
                def _():
                    idx_wait(scn, dcn, isemn)

                for t, (bb, ss) in enumerate(((b0, s0), (b1, s1))):
                    jj = 14 + t
                    pltpu.make_async_copy(h_hbm.at[sc.at[jj]], bb, ss).wait()
                    pltpu.sync_copy(bb, acc.at[dc.at[jj]], add=True)

                    @pl.when(g + 1 < NG)
                    def _():
                        pltpu.async_copy(h_hbm.at[scn.at[t]], bb, ss)

                @pl.when(g + 2 < NG)
                def _():
                    idx_start(g + 2, sc, dc, isem)

            def two_groups(p, carry):
                do_group(2 * p, sc0, dc0, i0, sc1, dc1, i1)
                do_group(2 * p + 1, sc1, dc1, i1, sc0, dc0, i0)
                return carry

            lax.fori_loop(0, NG // 2, two_groups, 0)
            plsc.subcore_barrier()
            for kk in range(ACC_R // LANE // 16):
                blkid = s * (ACC_R // LANE // 16) + kk
                r0 = blkid * LANE

                @pl.when(blkid < N // LANE)
                def _():
                    pltpu.sync_copy(acc.at[pl.ds(r0, LANE)], b0)
                    pltpu.sync_copy(b0, out_hbm.at[pl.ds(r0, LANE)])

                @pl.when(blkid == N // LANE)
                def _():
                    rem = N - (N // LANE) * LANE
                    pltpu.sync_copy(acc.at[pl.ds(r0, rem)], b0.at[pl.ds(0, rem)])
                    pltpu.sync_copy(b0.at[pl.ds(0, rem)],
                                    out_hbm.at[pl.ds(r0, rem)])

        @pl.when(c == 0)
        def _():
            run(ha_hbm, outa)

        @pl.when(c == 1)
        def _():
            run(hb_hbm, outb)

    return k(e3d, ha, hb)


def _tc_a(degT, x, W1):
    def body(dg, xr, w, dinv_r, ha_r, hb_r):
        deg = dg[:, 0:1] + dg[:, 1:2] + 1.0
        dinv = lax.rsqrt(deg)
        h = jnp.dot(xr[...], w[...], preferred_element_type=jnp.float32) * dinv
        dinv_r[...] = dinv
        ha_r[...] = h[:, :F]
        hb_r[...] = h[:, F:]

    return pl.pallas_call(
        body,
        grid=(N // RB,),
        in_specs=[
            pl.BlockSpec((RB, 2), lambda r: (r, 0)),
            pl.BlockSpec((RB, 128), lambda r: (r, 0)),
            pl.BlockSpec((128, H), lambda r: (0, 0)),
        ],
        out_specs=[
            pl.BlockSpec((RB, 1), lambda r: (r, 0)),
            pl.BlockSpec((RB, F), lambda r: (r, 0)),
            pl.BlockSpec((RB, F), lambda r: (r, 0)),
        ],
        out_shape=[
            jax.ShapeDtypeStruct((N, 1), jnp.float32),
            jax.ShapeDtypeStruct((N, F), jnp.float32),
            jax.ShapeDtypeStruct((N, F), jnp.float32),
        ],
    )(degT, x, W1)


def _tc_b(a1a, a1b, dinv, b1r, W2):
    def body(aa, ab, dv, br, w, h1a_r, h1b_r, ga_r, gb_r):
        agg = jnp.concatenate([aa[...], ab[...]], axis=1)
        dinv_b = dv[...]
        h1 = jnp.maximum(dinv_b * agg + br[...], 0.0)
        g = jnp.dot(h1, w[...], preferred_element_type=jnp.float32) * dinv_b
        h1a_r[...] = h1[:, :F]
        h1b_r[...] = h1[:, F:]
        ga_r[...] = g[:, :F]
        gb_r[...] = g[:, F:]

    return pl.pallas_call(
        body,
        grid=(N // RB,),
        in_specs=[
            pl.BlockSpec((RB, F), lambda r: (r, 0)),
            pl.BlockSpec((RB, F), lambda r: (r, 0)),
            pl.BlockSpec((RB, 1), lambda r: (r, 0)),
            pl.BlockSpec((1, H), lambda r: (0, 0)),
            pl.BlockSpec((H, H), lambda r: (0, 0)),
        ],
        out_specs=[
            pl.BlockSpec((RB, F), lambda r: (r, 0)),
            pl.BlockSpec((RB, F), lambda r: (r, 0)),
            pl.BlockSpec((RB, F), lambda r: (r, 0)),
            pl.BlockSpec((RB, F), lambda r: (r, 0)),
        ],
        out_shape=[
            jax.ShapeDtypeStruct((N, F), jnp.float32),
            jax.ShapeDtypeStruct((N, F), jnp.float32),
            jax.ShapeDtypeStruct((N, F), jnp.float32),
            jax.ShapeDtypeStruct((N, F), jnp.float32),
        ],
    )(a1a, a1b, dinv, b1r, W2)


def _tc_c(a2a, a2b, h1a, h1b, dinv, b2r, gam, bet):
    def body(aa, ab, ha, hb, dv, br, gm, bt, out_r):
        conv = jnp.concatenate([aa[...], ab[...]], axis=1)
        dinv_b = dv[...]
        h2 = (jnp.concatenate([ha[...], hb[...]], axis=1)
              + jnp.maximum(dinv_b * conv + br[...], 0.0))
        mu = jnp.mean(h2, axis=1, keepdims=True)
        d = h2 - mu
        var = jnp.mean(d * d, axis=1, keepdims=True)
        out_r[...] = gm[...] * d * lax.rsqrt(var + 1e-5) + bt[...]

    return pl.pallas_call(
        body,
        grid=(N // RB,),
        in_specs=[
            pl.BlockSpec((RB, F), lambda r: (r, 0)),
            pl.BlockSpec((RB, F), lambda r: (r, 0)),
            pl.BlockSpec((RB, F), lambda r: (r, 0)),
            pl.BlockSpec((RB, F), lambda r: (r, 0)),
            pl.BlockSpec((RB, 1), lambda r: (r, 0)),
            pl.BlockSpec((1, H), lambda r: (0, 0)),
            pl.BlockSpec((1, H), lambda r: (0, 0)),
            pl.BlockSpec((1, H), lambda r: (0, 0)),
        ],
        out_specs=pl.BlockSpec((RB, H), lambda r: (r, 0)),
        out_shape=jax.ShapeDtypeStruct((N, H), jnp.float32),
    )(a2a, a2b, h1a, h1b, dinv, b2r, gam, bet)


def kernel(x, edge_index, W1, b1, W2, b2, gamma, beta):
    ei = edge_index.astype(jnp.int32)
    pad = PE - E
    pad_src = jnp.arange(pad, dtype=jnp.int32) % 16
    pad_dst = N + jnp.arange(pad, dtype=jnp.int32) % 16
    e3d = jnp.concatenate(
        [ei, jnp.stack([pad_src, pad_dst])], axis=1).reshape(2, ROWS2D, LANE)

    degp = _sc_deg(e3d)
    degT = degp.reshape(2, DEG_R).T[:N]

    dinv, hs1a, hs1b = _tc_a(degT, x, W1)
    a1a, a1b = _sc_agg(e3d, hs1a, hs1b)
    h1a, h1b, g2a, g2b = _tc_b(a1a, a1b, dinv, b1.reshape(1, H), W2)
    a2a, a2b = _sc_agg(e3d, g2a, g2b)
    return _tc_c(a2a, a2b, h1a, h1b, dinv,
                 b2.reshape(1, H), gamma.reshape(1, H), beta.reshape(1, H))


# speedup vs baseline: 1.1182x; 1.0195x over previous
import functools

import jax
import jax.numpy as jnp
from jax import lax
from jax.experimental import pallas as pl
from jax.experimental.pallas import tpu as pltpu
from jax.experimental.pallas import tpu_sc as plsc

N = 10000
E = 320000
LANE = 128
CT = 160
PE = 16 * CT * LANE
ROWS2D = PE // LANE
DT = ROWS2D // 32
ACC_R = 10240
DEG_R = 10112
F = 128
H = 256
RB = 2000

_mesh = plsc.VectorSubcoreMesh(core_axis_name="c", subcore_axis_name="s")


def _sc_deg(e3d):
    @functools.partial(
        pl.kernel,
        mesh=_mesh,
        out_type=jax.ShapeDtypeStruct((2 * DEG_R,), jnp.float32),
        scratch_types=[
            pltpu.VMEM((DT, LANE), jnp.int32),
            pltpu.VMEM((LANE,), jnp.float32),
            pltpu.VMEM((LANE,), jnp.float32),
            pltpu.VMEM_SHARED((DEG_R,), jnp.float32),
            pltpu.SemaphoreType.DMA,
        ],
    )
    def k(e_hbm, out, dst_v, ones_v, zv, acc, t0):
        c = lax.axis_index("c")
        s = lax.axis_index("s")
        one = jnp.full((16,), 1.0, dtype=jnp.float32)
        zer = jnp.zeros((16,), dtype=jnp.float32)
        for j in range(LANE // 16):
            ones_v[pl.ds(16 * j, 16)] = one
            zv[pl.ds(16 * j, 16)] = zer
        for kk in range(5):
            idx = s * 5 + kk

            @pl.when(idx < DEG_R // LANE)
            def _():
                pltpu.sync_copy(zv, acc.at[pl.ds(idx * LANE, LANE)])

        plsc.subcore_barrier()
        wid = c * 16 + s
        pltpu.sync_copy(e_hbm.at[1, pl.ds(wid * DT, DT)], dst_v)

        def body(j, carry):
            pltpu.async_copy(ones_v, acc.at[dst_v.at[j]], t0, add=True)
            return carry

        lax.fori_loop(0, DT, body, 0)

        def drain(j, carry):
            pltpu.make_async_copy(ones_v, acc.at[dst_v.at[0]], t0).wait()
            return carry

        lax.fori_loop(0, DT, drain, 0)
        plsc.subcore_barrier()
        for kk in range(5):
            blkid = s * 5 + kk

            @pl.when(blkid < DEG_R // LANE)
            def _():
                r0 = blkid * LANE
                pltpu.sync_copy(acc.at[pl.ds(r0, LANE)], zv)
                pltpu.sync_copy(zv, out.at[pl.ds(c * DEG_R + r0, LANE)])

    return k(e3d)


def _sc_agg(e3d, ha, hb):
    @functools.partial(
        pl.kernel,
        mesh=_mesh,
        out_type=[
            jax.ShapeDtypeStruct((N, F), jnp.float32),
            jax.ShapeDtypeStruct((N, F), jnp.float32),
        ],
        scratch_types=[
            pltpu.VMEM((16, LANE), jnp.int32),
            pltpu.VMEM((16, LANE), jnp.int32),
            pltpu.VMEM((16, LANE), jnp.int32),
            pltpu.VMEM((16, LANE), jnp.int32),
            pltpu.VMEM((LANE, F), jnp.float32),
            pltpu.VMEM((LANE, F), jnp.float32),
            pltpu.VMEM_SHARED((ACC_R, F), jnp.float32),
            pltpu.SemaphoreType.DMA,
            pltpu.SemaphoreType.DMA,
            pltpu.SemaphoreType.DMA,
            pltpu.SemaphoreType.DMA,
        ],
    )
    def k(e_hbm, ha_hbm, hb_hbm, outa, outb,
          sc0, dc0, sc1, dc1, b0, b1, acc, s0, s1, i0, i1):
        c = lax.axis_index("c")
        s = lax.axis_index("s")
        NG = CT // 16

        def run(h_hbm, out_hbm):
            base = s * CT

            def idx_start(g, sc, dc, isem):
                pltpu.async_copy(
                    e_hbm.at[0, pl.ds(base + g * 16, 16)], sc, isem)
                pltpu.async_copy(
                    e_hbm.at[1, pl.ds(base + g * 16, 16)], dc, isem)

            def idx_wait(sc, dc, isem):
                pltpu.make_async_copy(
                    e_hbm.at[0, pl.ds(base, 16)], sc, isem).wait()
                pltpu.make_async_copy(
                    e_hbm.at[1, pl.ds(base, 16)], dc, isem).wait()

            idx_start(0, sc0, dc0, i0)
            idx_start(1, sc1, dc1, i1)

            NFB = N // LANE
            REM = N - NFB * LANE
            NBK = ACC_R // LANE // 16

            def iblk(kk):
                return s * NBK + kk, (s * NBK + kk) * LANE

            def ipull(kk, bb, ss):
                blkid, r0 = iblk(kk)

                @pl.when(blkid < NFB)
                def _():
                    pltpu.async_copy(h_hbm.at[pl.ds(r0, LANE)], bb, ss)

                @pl.when(blkid == NFB)
                def _():
                    pltpu.async_copy(h_hbm.at[pl.ds(r0, REM)],
                                     bb.at[pl.ds(0, REM)], ss)

            def ipullwait(kk, bb, ss):
                blkid, r0 = iblk(kk)

                @pl.when(blkid < NFB)
                def _():
                    pltpu.make_async_copy(
                        h_hbm.at[pl.ds(r0, LANE)], bb, ss).wait()

                @pl.when(blkid == NFB)
                def _():
                    pltpu.make_async_copy(
                        h_hbm.at[pl.ds(r0, REM)],
                        bb.at[pl.ds(0, REM)], ss).wait()

            def ipush(kk, bb):
                blkid, r0 = iblk(kk)

                @pl.when(blkid < NFB)
                def _():
                    pltpu.sync_copy(bb, acc.at[pl.ds(r0, LANE)])

                @pl.when(blkid == NFB)
                def _():
                    pltpu.sync_copy(bb.at[pl.ds(0, REM)],
                                    acc.at[pl.ds(r0, REM)])

            ipull(0, b0, s0)
            for kk in range(NBK):
                bb, ss = (b0, s0) if kk % 2 == 0 else (b1, s1)
                nb, ns = (b1, s1) if kk % 2 == 0 else (b0, s0)
                ipullwait(kk, bb, ss)
                if kk + 1 < NBK:
                    ipull(kk + 1, nb, ns)
                ipush(kk, bb)

            idx_wait(sc0, dc0, i0)
            pltpu.async_copy(h_hbm.at[sc0.at[0]], b0, s0)
            pltpu.async_copy(h_hbm.at[sc0.at[1]], b1, s1)
            plsc.subcore_barrier()

            def do_group(g, sc, dc, isem, scn, dcn, isemn):
                def pair(i, carry):
                    j = 2 * i
                    for t, (bb, ss) in enumerate(((b0, s0), (b1, s1))):
                        jj = j + t
                        pltpu.make_async_copy(
                            h_hbm.at[sc.at[jj]], bb, ss).wait()
                        pltpu.sync_copy(bb, acc.at[dc.at[jj]], add=True)
                        pltpu.async_copy(h_hbm.at[sc.at[jj + 2]], bb, ss)
                    return carry

                lax.fori_loop(0, 7, pair, 0)
                @pl.when(g + 1 < NG)
                def _():
                    idx_wait(scn, dcn, isemn)

                for t, (bb, ss) in enumerate(((b0, s0), (b1, s1))):
                    jj = 14 + t
                    pltpu.make_async_copy(h_hbm.at[sc.at[jj]], bb, ss).wait()
                    pltpu.sync_copy(bb, acc.at[dc.at[jj]], add=True)

                    @pl.when(g + 1 < NG)
                    def _():
                        pltpu.async_copy(h_hbm.at[scn.at[t]], bb, ss)

                @pl.when(g + 2 < NG)
                def _():
                    idx_start(g + 2, sc, dc, isem)

            def two_groups(p, carry):
                do_group(2 * p, sc0, dc0, i0, sc1, dc1, i1)
                do_group(2 * p + 1, sc1, dc1, i1, sc0, dc0, i0)
                return carry

            lax.fori_loop(0, NG // 2, two_groups, 0)
            plsc.subcore_barrier()

            def opush(kk, bb, ss):
                blkid, r0 = iblk(kk)

                @pl.when(blkid < NFB)
                def _():
                    pltpu.async_copy(bb, out_hbm.at[pl.ds(r0, LANE)], ss)

                @pl.when(blkid == NFB)
                def _():
                    pltpu.async_copy(bb.at[pl.ds(0, REM)],
                                     out_hbm.at[pl.ds(r0, REM)], ss)

            def opushwait(kk, bb, ss):
                blkid, r0 = iblk(kk)

                @pl.when(blkid < NFB)
                def _():
                    pltpu.make_async_copy(
                        bb, out_hbm.at[pl.ds(r0, LANE)], ss).wait()

                @pl.when(blkid == NFB)
                def _():
                    pltpu.make_async_copy(
                        bb.at[pl.ds(0, REM)],
                        out_hbm.at[pl.ds(r0, REM)], ss).wait()

            def opull(kk, bb):
                blkid, r0 = iblk(kk)

                @pl.when(blkid < NFB)
                def _():
                    pltpu.sync_copy(acc.at[pl.ds(r0, LANE)], bb)

                @pl.when(blkid == NFB)
                def _():
                    pltpu.sync_copy(acc.at[pl.ds(r0, REM)],
                                    bb.at[pl.ds(0, REM)])

            for kk in range(NBK):
                bb, ss = (b0, s0) if kk % 2 == 0 else (b1, s1)
                if kk >= 2:
                    opushwait(kk - 2, bb, ss)
                opull(kk, bb)
                opush(kk, bb, ss)
            opushwait(NBK - 2, b1 if (NBK - 2) % 2 else b0,
                      s1 if (NBK - 2) % 2 else s0)
            opushwait(NBK - 1, b1 if (NBK - 1) % 2 else b0,
                      s1 if (NBK - 1) % 2 else s0)

        @pl.when(c == 0)
        def _():
            run(ha_hbm, outa)

        @pl.when(c == 1)
        def _():
            run(hb_hbm, outb)

    return k(e3d, ha, hb)


def _tc_a(degT, x, W1):
    def body(dg, xr, w, dinv_r, ha_r, hb_r):
        deg = dg[:, 0:1] + dg[:, 1:2] + 1.0
        dinv = lax.rsqrt(deg)
        h = jnp.dot(xr[...], w[...], preferred_element_type=jnp.float32) * dinv
        dinv_r[...] = dinv
        ha_r[...] = h[:, :F]
        hb_r[...] = h[:, F:]

    return pl.pallas_call(
        body,
        grid=(N // RB,),
        in_specs=[
            pl.BlockSpec((RB, 2), lambda r: (r, 0)),
            pl.BlockSpec((RB, 128), lambda r: (r, 0)),
            pl.BlockSpec((128, H), lambda r: (0, 0)),
        ],
        out_specs=[
            pl.BlockSpec((RB, 1), lambda r: (r, 0)),
            pl.BlockSpec((RB, F), lambda r: (r, 0)),
            pl.BlockSpec((RB, F), lambda r: (r, 0)),
        ],
        out_shape=[
            jax.ShapeDtypeStruct((N, 1), jnp.float32),
            jax.ShapeDtypeStruct((N, F), jnp.float32),
            jax.ShapeDtypeStruct((N, F), jnp.float32),
        ],
    )(degT, x, W1)


def _tc_b(a1a, a1b, dinv, b1r, W2):
    def body(aa, ab, dv, br, w, h1a_r, h1b_r, ga_r, gb_r):
        agg = jnp.concatenate([aa[...], ab[...]], axis=1)
        dinv_b = dv[...]
        h1 = jnp.maximum(dinv_b * agg + br[...], 0.0)
        g = jnp.dot(h1, w[...], preferred_element_type=jnp.float32) * dinv_b
        h1a_r[...] = h1[:, :F]
        h1b_r[...] = h1[:, F:]
        ga_r[...] = g[:, :F]
        gb_r[...] = g[:, F:]

    return pl.pallas_call(
        body,
        grid=(N // RB,),
        in_specs=[
            pl.BlockSpec((RB, F), lambda r: (r, 0)),
            pl.BlockSpec((RB, F), lambda r: (r, 0)),
            pl.BlockSpec((RB, 1), lambda r: (r, 0)),
            pl.BlockSpec((1, H), lambda r: (0, 0)),
            pl.BlockSpec((H, H), lambda r: (0, 0)),
        ],
        out_specs=[
            pl.BlockSpec((RB, F), lambda r: (r, 0)),
            pl.BlockSpec((RB, F), lambda r: (r, 0)),
            pl.BlockSpec((RB, F), lambda r: (r, 0)),
            pl.BlockSpec((RB, F), lambda r: (r, 0)),
        ],
        out_shape=[
            jax.ShapeDtypeStruct((N, F), jnp.float32),
            jax.ShapeDtypeStruct((N, F), jnp.float32),
            jax.ShapeDtypeStruct((N, F), jnp.float32),
            jax.ShapeDtypeStruct((N, F), jnp.float32),
        ],
    )(a1a, a1b, dinv, b1r, W2)


def _tc_c(a2a, a2b, h1a, h1b, dinv, b2r, gam, bet):
    def body(aa, ab, ha, hb, dv, br, gm, bt, out_r):
        conv = jnp.concatenate([aa[...], ab[...]], axis=1)
        dinv_b = dv[...]
        h2 = (jnp.concatenate([ha[...], hb[...]], axis=1)
              + jnp.maximum(dinv_b * conv + br[...], 0.0))
        mu = jnp.mean(h2, axis=1, keepdims=True)
        d = h2 - mu
        var = jnp.mean(d * d, axis=1, keepdims=True)
        out_r[...] = gm[...] * d * lax.rsqrt(var + 1e-5) + bt[...]

    return pl.pallas_call(
        body,
        grid=(N // RB,),
        in_specs=[
            pl.BlockSpec((RB, F), lambda r: (r, 0)),
            pl.BlockSpec((RB, F), lambda r: (r, 0)),
            pl.BlockSpec((RB, F), lambda r: (r, 0)),
            pl.BlockSpec((RB, F), lambda r: (r, 0)),
            pl.BlockSpec((RB, 1), lambda r: (r, 0)),
            pl.BlockSpec((1, H), lambda r: (0, 0)),
            pl.BlockSpec((1, H), lambda r: (0, 0)),
            pl.BlockSpec((1, H), lambda r: (0, 0)),
        ],
        out_specs=pl.BlockSpec((RB, H), lambda r: (r, 0)),
        out_shape=jax.ShapeDtypeStruct((N, H), jnp.float32),
    )(a2a, a2b, h1a, h1b, dinv, b2r, gam, bet)


def kernel(x, edge_index, W1, b1, W2, b2, gamma, beta):
    ei = edge_index.astype(jnp.int32)
    pad = PE - E
    pad_src = jnp.arange(pad, dtype=jnp.int32) % 16
    pad_dst = N + jnp.arange(pad, dtype=jnp.int32) % 16
    e3d = jnp.concatenate(
        [ei, jnp.stack([pad_src, pad_dst])], axis=1).reshape(2, ROWS2D, LANE)

    degp = _sc_deg(e3d)
    degT = degp.reshape(2, DEG_R).T[:N]

    dinv, hs1a, hs1b = _tc_a(degT, x, W1)
    a1a, a1b = _sc_agg(e3d, hs1a, hs1b)
    h1a, h1b, g2a, g2b = _tc_b(a1a, a1b, dinv, b1.reshape(1, H), W2)
    a2a, a2b = _sc_agg(e3d, g2a, g2b)
    return _tc_c(a2a, a2b, h1a, h1b, dinv,
                 b2.reshape(1, H), gamma.reshape(1, H), beta.reshape(1, H))
